# baseline (speedup 1.0000x reference)
import functools

import jax
import jax.numpy as jnp
from jax import lax
from jax.experimental import pallas as pl
from jax.experimental.pallas import tpu as pltpu
from jax.experimental.pallas import tpu_sc as plsc

_MAX_REL = 128
_HEAD = 64
_VOCAB = 2 * _MAX_REL + 1
_L = 2048
_SAT = _L - 1 - _MAX_REL

_NC = 2
_NS = 16
_NW = _NC * _NS
_NPART = 4
_PART_ROWS = _L // _NPART
_ROWS_PER_W = _PART_ROWS // _NW
_W = 1024
_WIN = _W + _ROWS_PER_W - 1
_LANES = 16

_NPHASE = 8
_PROWS = 2 * _L


def _sc_body(key_hbm, out_k, tab_v, win_v, sem, *, part):
    wid = lax.axis_index("s") * _NC + lax.axis_index("c")
    r0 = part * _PART_ROWS + wid * _ROWS_PER_W

    pltpu.async_copy(key_hbm, tab_v, sem).wait()
    for j0 in (0, _W):
        p0 = (_L - _ROWS_PER_W) - r0 + j0

        def build_row(m, _, p0=p0):
            idx = jnp.clip(p0 + m - _SAT, 0, _VOCAB - 1)
            for c in range(_HEAD // _LANES):
                win_v[pl.ds(m * _HEAD + c * _LANES, _LANES)] = tab_v[
                    pl.ds(idx * _HEAD + c * _LANES, _LANES)
                ]
            return _

        lax.fori_loop(0, _WIN, build_row, 0, unroll=4)
        handles = [
            pltpu.async_copy(
                win_v.at[pl.ds((_ROWS_PER_W - 1 - k) * _HEAD, _W * _HEAD)],
                out_k.at[
                    pl.ds(((r0 + k - part * _PART_ROWS) * _L + j0) * _HEAD,
                          _W * _HEAD)
                ],
                sem,
            )
            for k in range(_ROWS_PER_W)
        ]
        for h in handles:
            h.wait()


def _tc_body(tab_ref, out_hbm, *rest):
    pals = rest[:_NPHASE]
    sem = rest[_NPHASE]
    tab = tab_ref[...]
    rows = lax.broadcasted_iota(jnp.int32, (_PROWS, _VOCAB), 0)
    cols = lax.broadcasted_iota(jnp.int32, (_PROWS, _VOCAB), 1)
    for f in range(_NPHASE):
        idx = jnp.clip(rows + (f - _SAT), 0, _VOCAB - 1)
        onehot = (idx == cols).astype(jnp.float32)
        pals[f][...] = jnp.dot(onehot, tab, preferred_element_type=jnp.float32,
                               precision=lax.Precision.HIGHEST)

    group = 8
    for f in range(_NPHASE):
        i0 = (_L - 1 - f) % _NPHASE
        nrows = _L // _NPHASE
        ngroups = nrows // group

        def copies_for(g, f=f, i0=i0):
            handles = []
            for u in range(group):
                t = g * group + u
                i = i0 + _NPHASE * t
                start = pl.multiple_of((_L - 1) - i - f, _NPHASE)
                handles.append(
                    pltpu.make_async_copy(
                        pals[f].at[pl.ds(start, _L), :],
                        out_hbm.at[pl.ds(i * _L, _L), :],
                        sem,
                    )
                )
            return handles

        for h in copies_for(0):
            h.start()

        def issue(g, _):
            nxt = copies_for(g + 1)
            for h in nxt:
                h.start()
            for h in copies_for(g):
                h.wait()
            return _

        lax.fori_loop(0, ngroups - 1, issue, 0)
        for h in copies_for(ngroups - 1):
            h.wait()


@jax.jit
def _rpe_call(rel_key_table, rel_value_table):
    mesh = plsc.VectorSubcoreMesh(core_axis_name="c", subcore_axis_name="s")
    key_parts = []
    for part in range(_NPART):
        sc_fn = functools.partial(
            pl.kernel,
            mesh=mesh,
            out_type=jax.ShapeDtypeStruct((_PART_ROWS * _L * _HEAD,), jnp.float32),
            scratch_types=[
                pltpu.VMEM((_VOCAB * _HEAD,), jnp.float32),
                pltpu.VMEM((_WIN * _HEAD,), jnp.float32),
                pltpu.SemaphoreType.DMA,
            ],
        )(functools.partial(_sc_body, part=part))
        key_parts.append(sc_fn(rel_key_table.reshape(_VOCAB * _HEAD)))

    out_v = pl.pallas_call(
        _tc_body,
        out_shape=jax.ShapeDtypeStruct((_L * _L, _HEAD), jnp.float32),
        in_specs=[pl.BlockSpec(memory_space=pltpu.VMEM)],
        out_specs=pl.BlockSpec(memory_space=pl.ANY),
        scratch_shapes=[pltpu.VMEM((_PROWS, _HEAD), jnp.float32)] * _NPHASE
        + [pltpu.SemaphoreType.DMA],
    )(rel_value_table)
    return key_parts, out_v


def kernel(length, rel_key_table, rel_value_table):
    key_parts, out_v = _rpe_call(rel_key_table, rel_value_table)
    out_k = jnp.concatenate(
        [p.reshape(_PART_ROWS, _L, _HEAD) for p in key_parts], axis=0
    )
    return (out_k, out_v.reshape(_L, _L, _HEAD))


# speedup vs baseline: 1.1101x; 1.1101x over previous
import functools

import jax
import jax.numpy as jnp
from jax import lax
from jax.experimental import pallas as pl
from jax.experimental.pallas import tpu as pltpu
from jax.experimental.pallas import tpu_sc as plsc

_MAX_REL = 128
_HEAD = 64
_VOCAB = 2 * _MAX_REL + 1
_L = 2048
_SAT = _L - 1 - _MAX_REL

_NC = 2
_NS = 16
_NW = _NC * _NS
_ROWS_PER_W = _L // _NW
_W = 1024
_WIN = _W + _ROWS_PER_W - 1
_LANES = 16

_NPHASE = 8
_PROWS = 2 * _L


def _sc_body(key_hbm, out_k, tab_v, win_v, sem):
    wid = lax.axis_index("s") * _NC + lax.axis_index("c")
    r0 = wid * _ROWS_PER_W

    pltpu.async_copy(key_hbm, tab_v, sem).wait()
    for j0 in (0, _W):
        p0 = (_L - _ROWS_PER_W) - r0 + j0

        def build_row(m, _, p0=p0):
            idx = jnp.clip(p0 + m - _SAT, 0, _VOCAB - 1)
            for c in range(_HEAD // _LANES):
                win_v[pl.ds(m * _HEAD + c * _LANES, _LANES)] = tab_v[
                    pl.ds(idx * _HEAD + c * _LANES, _LANES)
                ]
            return _

        lax.fori_loop(0, _WIN, build_row, 0, unroll=4)
        handles = [
            pltpu.async_copy(
                win_v.at[pl.ds((_ROWS_PER_W - 1 - k) * _HEAD, _W * _HEAD)],
                out_k.at[pl.ds(((r0 + k) * _L + j0) * _HEAD, _W * _HEAD)],
                sem,
            )
            for k in range(_ROWS_PER_W)
        ]
        for h in handles:
            h.wait()


def _tc_body(tab_ref, out_hbm, *rest):
    pals = rest[:_NPHASE]
    sem = rest[_NPHASE]
    tab = tab_ref[...]
    rows = lax.broadcasted_iota(jnp.int32, (_PROWS, _VOCAB), 0)
    cols = lax.broadcasted_iota(jnp.int32, (_PROWS, _VOCAB), 1)
    for f in range(_NPHASE):
        idx = jnp.clip(rows + (f - _SAT), 0, _VOCAB - 1)
        onehot = (idx == cols).astype(jnp.float32)
        pals[f][...] = jnp.dot(onehot, tab, preferred_element_type=jnp.float32,
                               precision=lax.Precision.HIGHEST)

    group = 8
    for f in range(_NPHASE):
        i0 = (_L - 1 - f) % _NPHASE
        nrows = _L // _NPHASE

        def issue(g, _, f=f, i0=i0):
            handles = []
            for u in range(group):
                t = g * group + u
                i = i0 + _NPHASE * t
                start = pl.multiple_of((_L - 1) - i - f, _NPHASE)
                handles.append(
                    pltpu.make_async_copy(
                        pals[f].at[pl.ds(start, _L), :],
                        out_hbm.at[i],
                        sem,
                    )
                )
            for h in handles:
                h.start()
            for h in handles:
                h.wait()
            return _

        lax.fori_loop(0, nrows // group, issue, 0)


@jax.jit
def _rpe_call(rel_key_table, rel_value_table):
    mesh = plsc.VectorSubcoreMesh(core_axis_name="c", subcore_axis_name="s")
    sc_fn = functools.partial(
        pl.kernel,
        mesh=mesh,
        out_type=jax.ShapeDtypeStruct((_L * _L * _HEAD,), jnp.float32),
        scratch_types=[
            pltpu.VMEM((_VOCAB * _HEAD,), jnp.float32),
            pltpu.VMEM((_WIN * _HEAD,), jnp.float32),
            pltpu.SemaphoreType.DMA,
        ],
    )(_sc_body)
    out_v = sc_fn(rel_value_table.reshape(_VOCAB * _HEAD))

    out_k = pl.pallas_call(
        _tc_body,
        out_shape=jax.ShapeDtypeStruct((_L, _L, _HEAD), jnp.float32),
        in_specs=[pl.BlockSpec(memory_space=pltpu.VMEM)],
        out_specs=pl.BlockSpec(memory_space=pl.ANY),
        scratch_shapes=[pltpu.VMEM((_PROWS, _HEAD), jnp.float32)] * _NPHASE
        + [pltpu.SemaphoreType.DMA],
    )(rel_key_table)
    return out_k, out_v


def kernel(length, rel_key_table, rel_value_table):
    out_k, out_v = _rpe_call(rel_key_table, rel_value_table)
    return (out_k, out_v.reshape(_L, _L, _HEAD))


# speedup vs baseline: 1.3874x; 1.2498x over previous
import functools

import jax
import jax.numpy as jnp
from jax import lax
from jax.experimental import pallas as pl
from jax.experimental.pallas import tpu as pltpu
from jax.experimental.pallas import tpu_sc as plsc

_MAX_REL = 128
_HEAD = 64
_VOCAB = 2 * _MAX_REL + 1
_L = 2048
_SAT = _L - 1 - _MAX_REL

_NC = 2
_NS = 16
_NW = _NC * _NS
_ROWS_PER_W = _L // _NW
_W = 1024
_WIN = _W + _ROWS_PER_W - 1
_LANES = 16

_NPHASE = 8
_PROWS = 2 * _L


def _sc_body(key_hbm, out_k, tab_v, win_v, sem):
    wid = lax.axis_index("s") * _NC + lax.axis_index("c")
    r0 = wid * _ROWS_PER_W

    pltpu.async_copy(key_hbm, tab_v, sem).wait()
    for j0 in (0, _W):
        p0 = (_L - _ROWS_PER_W) - r0 + j0

        def build_row(m, _, p0=p0):
            idx = jnp.clip(p0 + m - _SAT, 0, _VOCAB - 1)
            for c in range(_HEAD // _LANES):
                win_v[pl.ds(m * _HEAD + c * _LANES, _LANES)] = tab_v[
                    pl.ds(idx * _HEAD + c * _LANES, _LANES)
                ]
            return _

        lax.fori_loop(0, _WIN, build_row, 0, unroll=4)
        handles = [
            pltpu.async_copy(
                win_v.at[pl.ds((_ROWS_PER_W - 1 - k) * _HEAD, _W * _HEAD)],
                out_k.at[pl.ds(((r0 + k) * _L + j0) * _HEAD, _W * _HEAD)],
                sem,
            )
            for k in range(_ROWS_PER_W)
        ]
        for h in handles:
            h.wait()


def _tc_body(tab_ref, out_hbm, *rest):
    pals = rest[:_NPHASE]
    sem = rest[_NPHASE]
    tab = tab_ref[...]
    rows = lax.broadcasted_iota(jnp.int32, (_PROWS, _VOCAB), 0)
    cols = lax.broadcasted_iota(jnp.int32, (_PROWS, _VOCAB), 1)
    for f in range(_NPHASE):
        idx = jnp.clip(rows + (f - _SAT), 0, _VOCAB - 1)
        onehot = (idx == cols).astype(jnp.float32)
        pals[f][...] = jnp.dot(onehot, tab, preferred_element_type=jnp.float32,
                               precision=lax.Precision.HIGHEST)

    group = 8
    for f in range(_NPHASE):
        i0 = (_L - 1 - f) % _NPHASE
        nrows = _L // _NPHASE

        def issue(g, _, f=f, i0=i0):
            handles = []
            for u in range(group):
                t = g * group + u
                i = i0 + _NPHASE * t
                start = pl.multiple_of((_L - 1) - i - f, _NPHASE)
                handles.append(
                    pltpu.make_async_copy(
                        pals[f].at[pl.ds(start, _L), :],
                        out_hbm.at[pl.ds(i * _L, _L), :],
                        sem,
                    )
                )
            for h in handles:
                h.start()
            for h in handles:
                h.wait()
            return _

        lax.fori_loop(0, nrows // group, issue, 0)


@jax.jit
def _rpe_call(rel_key_table, rel_value_table):
    mesh = plsc.VectorSubcoreMesh(core_axis_name="c", subcore_axis_name="s")
    sc_fn = functools.partial(
        pl.kernel,
        mesh=mesh,
        out_type=jax.ShapeDtypeStruct((_L * _L * _HEAD,), jnp.float32),
        scratch_types=[
            pltpu.VMEM((_VOCAB * _HEAD,), jnp.float32),
            pltpu.VMEM((_WIN * _HEAD,), jnp.float32),
            pltpu.SemaphoreType.DMA,
        ],
    )(_sc_body)
    out_k = sc_fn(rel_key_table.reshape(_VOCAB * _HEAD))

    out_v = pl.pallas_call(
        _tc_body,
        out_shape=jax.ShapeDtypeStruct((_L * _L, _HEAD), jnp.float32),
        in_specs=[pl.BlockSpec(memory_space=pltpu.VMEM)],
        out_specs=pl.BlockSpec(memory_space=pl.ANY),
        scratch_shapes=[pltpu.VMEM((_PROWS, _HEAD), jnp.float32)] * _NPHASE
        + [pltpu.SemaphoreType.DMA],
    )(rel_value_table)
    return out_k, out_v


def kernel(length, rel_key_table, rel_value_table):
    out_k, out_v = _rpe_call(rel_key_table, rel_value_table)
    return (
        out_k.reshape(_L, _L, _HEAD),
        out_v.reshape(_L, _L, _HEAD),
    )
